# baseline (speedup 1.0000x reference)
import functools

import jax
import jax.numpy as jnp
from jax import lax
from jax.experimental import pallas as pl
from jax.experimental.pallas import tpu as pltpu
from jax.experimental.pallas import tpu_sc as plsc

B = 256
BC = 64
NCHUNK = B // BC
NC, NS = 2, 16
NW = NC * NS
UNROLL = 16


@functools.lru_cache(maxsize=None)
def _make_layer(I, O):
    npw = O // NW
    mesh = plsc.VectorSubcoreMesh(core_axis_name="c", subcore_axis_name="s",
                                  num_cores=NC, num_subcores=NS)

    @functools.partial(
        pl.kernel,
        out_type=jax.ShapeDtypeStruct((NCHUNK, O, BC), jnp.uint16),
        mesh=mesh,
        compiler_params=pltpu.CompilerParams(use_tc_tiling_on_sc=False),
        scratch_types=[
            pltpu.VMEM((2 * I, BC), jnp.uint16),
            pltpu.VMEM((npw * I, 32), jnp.uint16),
            pltpu.VMEM((npw * I,), jnp.int32),
            pltpu.VMEM((2, 16), jnp.int32),
            pltpu.VMEM((npw, BC), jnp.uint16),
            pltpu.SemaphoreType.DMA,
        ],
    )
    def layer(z_hbm, idx_hbm, tab_hbm, base_hbm, out_hbm,
              z_v, m_v, i_v, b_v, o_v, sem):
        c = lax.axis_index("c")
        s = lax.axis_index("s")
        w = s * NC + c
        pltpu.sync_copy(idx_hbm.at[pl.ds(w * npw * I, npw * I)], i_v)
        pltpu.sync_copy(base_hbm.at[w], b_v)
        pltpu.async_copy(tab_hbm.at[i_v], m_v, sem).wait()
        bvec = b_v[0, :]
        zero = jnp.zeros((32,), jnp.uint16)

        def chunk(ci, carry):
            pltpu.sync_copy(z_hbm.at[ci], z_v)

            for op in range(npw // 2):
                o0 = 2 * op
                o1 = o0 + 1
                base0 = bvec[o0]
                base1 = bvec[o1]

                @pl.when(base0 == base1)
                def _(o0=o0, o1=o1, base=base0):
                    def body(iu, accs):
                        a00, a01, a10, a11 = accs
                        i0 = iu * UNROLL
                        for u in range(UNROLL):
                            r = base + i0 + u
                            z0 = z_v[r, pl.ds(0, 32)]
                            z1 = z_v[r, pl.ds(32, 32)]
                            m0 = m_v[o0 * I + i0 + u, :]
                            m1 = m_v[o1 * I + i0 + u, :]
                            a00 = jnp.maximum(a00, z0 & m0)
                            a01 = jnp.maximum(a01, z1 & m0)
                            a10 = jnp.maximum(a10, z0 & m1)
                            a11 = jnp.maximum(a11, z1 & m1)
                        return a00, a01, a10, a11

                    a00, a01, a10, a11 = lax.fori_loop(
                        0, I // UNROLL, body, (zero, zero, zero, zero))
                    o_v[o0, pl.ds(0, 32)] = a00
                    o_v[o0, pl.ds(32, 32)] = a01
                    o_v[o1, pl.ds(0, 32)] = a10
                    o_v[o1, pl.ds(32, 32)] = a11

                @pl.when(base0 != base1)
                def _(o0=o0, o1=o1, base0=base0, base1=base1):
                    def body(iu, accs):
                        a00, a01, a10, a11 = accs
                        i0 = iu * UNROLL
                        for u in range(UNROLL):
                            z0 = z_v[base0 + i0 + u, pl.ds(0, 32)]
                            z1 = z_v[base0 + i0 + u, pl.ds(32, 32)]
                            y0 = z_v[base1 + i0 + u, pl.ds(0, 32)]
                            y1 = z_v[base1 + i0 + u, pl.ds(32, 32)]
                            m0 = m_v[o0 * I + i0 + u, :]
                            m1 = m_v[o1 * I + i0 + u, :]
                            a00 = jnp.maximum(a00, z0 & m0)
                            a01 = jnp.maximum(a01, z1 & m0)
                            a10 = jnp.maximum(a10, y0 & m1)
                            a11 = jnp.maximum(a11, y1 & m1)
                        return a00, a01, a10, a11

                    a00, a01, a10, a11 = lax.fori_loop(
                        0, I // UNROLL, body, (zero, zero, zero, zero))
                    o_v[o0, pl.ds(0, 32)] = a00
                    o_v[o0, pl.ds(32, 32)] = a01
                    o_v[o1, pl.ds(0, 32)] = a10
                    o_v[o1, pl.ds(32, 32)] = a11

            pltpu.sync_copy(o_v, out_hbm.at[ci, pl.ds(w * npw, npw), :])
            return carry

        lax.fori_loop(0, NCHUNK, chunk, 0)

    return layer


def _node_params(ntc, g, I):
    logits = ntc + g
    y_soft = jax.nn.softmax(logits, axis=-1)
    amax = jnp.argmax(logits, axis=-1)
    y_hard = jax.nn.one_hot(amax, 2, dtype=logits.dtype)
    sel = y_soft + (y_hard - y_soft)
    is_max = amax == 1
    base = jnp.where(is_max, 0, I).astype(jnp.int32)
    p = jnp.where(is_max, 0.0, sel[:, 0])
    q = jnp.where(is_max, sel[:, 1], -sel[:, 0])
    return base, p, q


def _pack_worker(a, npw, width=16):
    a = a.reshape(NW, npw)
    pad = jnp.zeros((NW, width - npw), a.dtype)
    return jnp.concatenate([a, pad], axis=1)


def _partition_perm(is_max):
    t = is_max.astype(jnp.int32)
    O = t.shape[0]
    n0 = jnp.sum(1 - t)
    rank0 = jnp.cumsum(1 - t) - 1
    rank1 = n0 + jnp.cumsum(t) - 1
    pos = jnp.where(t == 0, rank0, rank1).astype(jnp.int32)
    perm = jnp.zeros((O,), jnp.int32).at[pos].set(
        jnp.arange(O, dtype=jnp.int32))
    return perm, pos


def _pack_base_flag(base, npw):
    b = base.reshape(NW, npw)
    flag = jnp.all(b[:, 0::2] == b[:, 1::2], axis=1).astype(jnp.int32)
    bp = jnp.concatenate([b, jnp.zeros((NW, 16 - npw), jnp.int32)], axis=1)
    fp = jnp.concatenate([flag[:, None],
                          jnp.zeros((NW, 15), jnp.int32)], axis=1)
    return jnp.stack([bp, fp], axis=1)


_TABLE = None
_TSPREAD = 2048


def _mask_table():
    global _TABLE
    if _TABLE is None:
        import numpy as np
        t = np.zeros((2 * _TSPREAD, 32), np.uint16)
        t[1::2, :] = 0xFFFF
        _TABLE = jnp.asarray(t)
    return _TABLE


def _mask_idx(noedge):
    O, I = noedge.shape
    lin = jnp.arange(O * I, dtype=jnp.int32).reshape(O, I)
    spread = (lin % _TSPREAD) * 2
    return (spread + jnp.where(noedge, 0, 1)).astype(jnp.int32).reshape(O * I)


def _to_u16(f):
    return lax.bitcast_convert_type(f.astype(jnp.bfloat16), jnp.uint16)


def _chunked(u):
    rows = u.shape[0]
    return u.reshape(rows, NCHUNK, BC).transpose(1, 0, 2)


def _post(red_u16, p, q):
    red = lax.bitcast_convert_type(red_u16, jnp.bfloat16).astype(jnp.float32)
    return p[None, :, None] + q[None, :, None] * red


def kernel(x, ntc1, ntc2, g1, g2, noedge1, noedge2):
    b1, p1, q1 = _node_params(ntc1, g1, 512)
    b2, p2, q2 = _node_params(ntc2, g2, 256)
    tab = _mask_table()

    perm1, _ = _partition_perm(b1 != 0)
    perm2, pos2 = _partition_perm(b2 != 0)
    i1 = _mask_idx(noedge1[perm1, :])
    i2 = _mask_idx(noedge2[perm2][:, perm1])

    z1 = _chunked(_to_u16(jnp.concatenate([x.T, (1.0 - x).T], axis=0)))
    r1 = _make_layer(512, 256)(z1, i1, tab,
                               _pack_base_flag(b1[perm1], 256 // NW))
    h = _post(r1, p1[perm1], q1[perm1])

    hh = jnp.concatenate([h, 1.0 - h], axis=1)
    z2 = _to_u16(hh)
    r2 = _make_layer(256, 512)(z2, i2, tab,
                               _pack_base_flag(b2[perm2], 512 // NW))
    out = _post(r2, p2[perm2], q2[perm2])
    out = jnp.take(out, pos2, axis=1)
    return out.transpose(0, 2, 1).reshape(B, 512)


# speedup vs baseline: 1.0782x; 1.0782x over previous
import functools

import jax
import jax.numpy as jnp
from jax import lax
from jax.experimental import pallas as pl
from jax.experimental.pallas import tpu as pltpu
from jax.experimental.pallas import tpu_sc as plsc

B = 256
BC = 64
NCHUNK = B // BC
NC, NS = 2, 16
NW = NC * NS
UNROLL = 16


@functools.lru_cache(maxsize=None)
def _make_layer(I, O):
    npw = O // NW
    mesh = plsc.VectorSubcoreMesh(core_axis_name="c", subcore_axis_name="s",
                                  num_cores=NC, num_subcores=NS)

    @functools.partial(
        pl.kernel,
        out_type=jax.ShapeDtypeStruct((NCHUNK, O, BC), jnp.uint16),
        mesh=mesh,
        compiler_params=pltpu.CompilerParams(use_tc_tiling_on_sc=False),
        scratch_types=[
            pltpu.VMEM((2 * I, BC), jnp.uint16),
            pltpu.VMEM((npw * I, 32), jnp.uint16),
            pltpu.VMEM((npw * I,), jnp.int32),
            pltpu.VMEM((16,), jnp.int32),
            pltpu.VMEM((npw, BC), jnp.uint16),
            pltpu.SemaphoreType.DMA,
        ],
    )
    def layer(z_hbm, idx_hbm, tab_hbm, base_hbm, out_hbm,
              z_v, m_v, i_v, b_v, o_v, sem):
        c = lax.axis_index("c")
        s = lax.axis_index("s")
        w = s * NC + c
        pltpu.sync_copy(idx_hbm.at[pl.ds(w * npw * I, npw * I)], i_v)
        pltpu.sync_copy(base_hbm.at[w], b_v)
        pltpu.async_copy(tab_hbm.at[i_v], m_v, sem).wait()
        bvec = b_v[...]

        def chunk(ci, carry):
            pltpu.sync_copy(z_hbm.at[ci], z_v)
            for o in range(npw):
                base = bvec[o]

                def body(iu, accs, o=o, base=base):
                    a0, a1 = accs
                    i0 = iu * UNROLL
                    for u in range(UNROLL):
                        mm = m_v[o * I + i0 + u, :]
                        z0 = z_v[base + i0 + u, pl.ds(0, 32)]
                        z1 = z_v[base + i0 + u, pl.ds(32, 32)]
                        a0 = jnp.maximum(a0, z0 & mm)
                        a1 = jnp.maximum(a1, z1 & mm)
                    return a0, a1

                zero = jnp.zeros((32,), jnp.uint16)
                a0, a1 = lax.fori_loop(0, I // UNROLL, body, (zero, zero))
                o_v[o, pl.ds(0, 32)] = a0
                o_v[o, pl.ds(32, 32)] = a1
            pltpu.sync_copy(o_v, out_hbm.at[ci, pl.ds(w * npw, npw), :])
            return carry

        lax.fori_loop(0, NCHUNK, chunk, 0)

    return layer


def _node_params(ntc, g, I):
    logits = ntc + g
    y_soft = jax.nn.softmax(logits, axis=-1)
    amax = jnp.argmax(logits, axis=-1)
    y_hard = jax.nn.one_hot(amax, 2, dtype=logits.dtype)
    sel = y_soft + (y_hard - y_soft)
    is_max = amax == 1
    base = jnp.where(is_max, 0, I).astype(jnp.int32)
    p = jnp.where(is_max, 0.0, sel[:, 0])
    q = jnp.where(is_max, sel[:, 1], -sel[:, 0])
    return base, p, q


def _pack_worker(a, npw, width=16):
    a = a.reshape(NW, npw)
    pad = jnp.zeros((NW, width - npw), a.dtype)
    return jnp.concatenate([a, pad], axis=1)


_TABLE = None
_TSPREAD = 2048


def _mask_table():
    global _TABLE
    if _TABLE is None:
        import numpy as np
        t = np.zeros((2 * _TSPREAD, 32), np.uint16)
        t[1::2, :] = 0xFFFF
        _TABLE = jnp.asarray(t)
    return _TABLE


def _mask_idx(noedge):
    O, I = noedge.shape
    lin = jnp.arange(O * I, dtype=jnp.int32).reshape(O, I)
    spread = (lin % _TSPREAD) * 2
    return (spread + jnp.where(noedge, 0, 1)).astype(jnp.int32).reshape(O * I)


def _to_u16(f):
    return lax.bitcast_convert_type(f.astype(jnp.bfloat16), jnp.uint16)


def _chunked(u):
    rows = u.shape[0]
    return u.reshape(rows, NCHUNK, BC).transpose(1, 0, 2)


def _post(red_u16, p, q):
    red = lax.bitcast_convert_type(red_u16, jnp.bfloat16).astype(jnp.float32)
    return p[None, :, None] + q[None, :, None] * red


def kernel(x, ntc1, ntc2, g1, g2, noedge1, noedge2):
    b1, p1, q1 = _node_params(ntc1, g1, 512)
    b2, p2, q2 = _node_params(ntc2, g2, 256)
    tab = _mask_table()
    i1 = _mask_idx(noedge1)
    i2 = _mask_idx(noedge2)

    z1 = _chunked(_to_u16(jnp.concatenate([x.T, (1.0 - x).T], axis=0)))
    r1 = _make_layer(512, 256)(z1, i1, tab, _pack_worker(b1, 256 // NW))
    h = _post(r1, p1, q1)

    hh = jnp.concatenate([h, 1.0 - h], axis=1)
    z2 = _to_u16(hh)
    r2 = _make_layer(256, 512)(z2, i2, tab, _pack_worker(b2, 512 // NW))
    out = _post(r2, p2, q2)
    return out.transpose(0, 2, 1).reshape(B, 512)
